# baseline (speedup 1.0000x reference)
import functools

import jax
import jax.numpy as jnp
import numpy as np
from jax.experimental import pallas as pl
from jax.experimental.pallas import tpu as pltpu

_BS = 1024


def _xw_kernel(x_ref, w_ref, o_ref):
    o_ref[...] = jnp.dot(x_ref[...], w_ref[...],
                         preferred_element_type=jnp.float32)


def _tri_kernel(im_ref, jm_ref, adj_ref, x1_ref, b1_ref, w2_ref, b2_ref,
                out_ref, y_ref, z_ref, *, nb, n):
    t = pl.program_id(0)
    i = im_ref[t]
    j = jm_ref[t]
    sweep1 = t < nb * nb
    rows = pl.ds(i * _BS, _BS)
    kb = n - (nb - 1) * _BS

    @pl.when(j == nb - 1)
    def _():
        adj_ref[:, kb:] = jnp.zeros((_BS, _BS - kb), jnp.float32)

    tile = adj_ref[...]

    @pl.when(t == 0)
    def _():
        out_ref[...] = jnp.zeros_like(out_ref)

    @pl.when(sweep1)
    def _():
        y = jnp.dot(tile, x1_ref[pl.ds(j * _BS, _BS), :],
                    preferred_element_type=jnp.float32)

        @pl.when(j == 0)
        def _():
            y_ref[rows, :] = y

        @pl.when(j > 0)
        def _():
            y_ref[rows, :] = y_ref[rows, :] + y

        @pl.when(j < i)
        def _():
            out_ref[rows, :] = out_ref[rows, :] + jnp.dot(
                tile, z_ref[pl.ds(j * _BS, _BS), :],
                preferred_element_type=jnp.float32)

        @pl.when(j == nb - 1)
        def _():
            h = jnp.maximum(y_ref[rows, :] + b1_ref[...], 0.0)
            z = jnp.dot(h, w2_ref[...], preferred_element_type=jnp.float32)
            ridx = i * _BS + jax.lax.broadcasted_iota(jnp.int32, z.shape, 0)
            z_ref[rows, :] = jnp.where(ridx < n, z, 0.0)

    @pl.when(jnp.logical_not(sweep1))
    def _():
        acc = out_ref[rows, :] + jnp.dot(
            tile, z_ref[pl.ds(j * _BS, _BS), :],
            preferred_element_type=jnp.float32)

        @pl.when(j < nb - 1)
        def _():
            out_ref[rows, :] = acc

        @pl.when(j == nb - 1)
        def _():
            o = acc + b2_ref[...]
            m = jnp.max(o, axis=1, keepdims=True)
            e = o - m
            out_ref[rows, :] = e - jnp.log(
                jnp.sum(jnp.exp(e), axis=1, keepdims=True))


@jax.jit
def kernel(feature, adj, W1, b1, W2, b2):
    n, f_in = feature.shape
    h_dim = W1.shape[1]
    c_dim = W2.shape[1]
    nb = -(-n // _BS)
    npad = nb * _BS

    im1, jm1 = np.divmod(np.arange(nb * nb, dtype=np.int32), nb)
    iu, ju = np.triu_indices(nb)
    im = jnp.asarray(np.concatenate([im1, iu.astype(np.int32)]))
    jm = jnp.asarray(np.concatenate([jm1, ju.astype(np.int32)]))
    steps = int(im.shape[0])

    feature_p = jnp.pad(feature, ((0, npad - n), (0, 0)))
    x1 = pl.pallas_call(
        _xw_kernel,
        grid=(nb,),
        in_specs=[
            pl.BlockSpec((_BS, f_in), lambda i: (i, 0)),
            pl.BlockSpec((f_in, h_dim), lambda i: (0, 0)),
        ],
        out_specs=pl.BlockSpec((_BS, h_dim), lambda i: (i, 0)),
        out_shape=jax.ShapeDtypeStruct((npad, h_dim), jnp.float32),
    )(feature_p, W1)

    b1_2d = b1.reshape(1, h_dim)
    b2_2d = b2.reshape(1, c_dim)

    grid_spec = pltpu.PrefetchScalarGridSpec(
        num_scalar_prefetch=2,
        grid=(steps,),
        in_specs=[
            pl.BlockSpec((_BS, _BS), lambda t, im_, jm_: (im_[t], jm_[t])),
            pl.BlockSpec((npad, h_dim), lambda t, im_, jm_: (0, 0)),
            pl.BlockSpec((1, h_dim), lambda t, im_, jm_: (0, 0)),
            pl.BlockSpec((h_dim, c_dim), lambda t, im_, jm_: (0, 0)),
            pl.BlockSpec((1, c_dim), lambda t, im_, jm_: (0, 0)),
        ],
        out_specs=pl.BlockSpec((npad, c_dim), lambda t, im_, jm_: (0, 0)),
        scratch_shapes=[
            pltpu.VMEM((npad, h_dim), jnp.float32),
            pltpu.VMEM((npad, c_dim), jnp.float32),
        ],
    )

    out = pl.pallas_call(
        functools.partial(_tri_kernel, nb=nb, n=n),
        grid_spec=grid_spec,
        out_shape=jax.ShapeDtypeStruct((npad, c_dim), jnp.float32),
    )(im, jm, adj, x1, b1_2d, W2, b2_2d)

    return out[:n]


# speedup vs baseline: 1.0352x; 1.0352x over previous
import functools

import jax
import jax.numpy as jnp
import numpy as np
from jax.experimental import pallas as pl
from jax.experimental.pallas import tpu as pltpu

_BS = 1024


def _xw_kernel(x_ref, w_ref, o_ref):
    o_ref[...] = jnp.dot(x_ref[...], w_ref[...],
                         preferred_element_type=jnp.float32
                         ).astype(jnp.bfloat16)


def _tri_kernel(im_ref, jm_ref, adj_ref, x1_ref, b1_ref, w2_ref, b2_ref,
                out_ref, y_ref, z_ref, *, nb, n):
    t = pl.program_id(0)
    i = im_ref[t]
    j = jm_ref[t]
    sweep1 = t < nb * nb
    rows = pl.ds(i * _BS, _BS)
    kb = n - (nb - 1) * _BS

    @pl.when(j == nb - 1)
    def _():
        adj_ref[:, kb:] = jnp.zeros((_BS, _BS - kb), jnp.float32)

    tile = adj_ref[...].astype(jnp.bfloat16)

    @pl.when(t == 0)
    def _():
        out_ref[...] = jnp.zeros_like(out_ref)

    @pl.when(sweep1)
    def _():
        y = jnp.dot(tile, x1_ref[pl.ds(j * _BS, _BS), :],
                    preferred_element_type=jnp.float32)

        @pl.when(j == 0)
        def _():
            y_ref[rows, :] = y

        @pl.when(j > 0)
        def _():
            y_ref[rows, :] = y_ref[rows, :] + y

        @pl.when(j < i)
        def _():
            out_ref[rows, :] = out_ref[rows, :] + jnp.dot(
                tile, z_ref[pl.ds(j * _BS, _BS), :],
                preferred_element_type=jnp.float32)

        @pl.when(j == nb - 1)
        def _():
            h = jnp.maximum(y_ref[rows, :] + b1_ref[...], 0.0)
            z = jnp.dot(h.astype(jnp.bfloat16), w2_ref[...],
                        preferred_element_type=jnp.float32)
            ridx = i * _BS + jax.lax.broadcasted_iota(jnp.int32, z.shape, 0)
            z_ref[rows, :] = jnp.where(ridx < n, z, 0.0).astype(jnp.bfloat16)

    @pl.when(jnp.logical_not(sweep1))
    def _():
        acc = out_ref[rows, :] + jnp.dot(
            tile, z_ref[pl.ds(j * _BS, _BS), :],
            preferred_element_type=jnp.float32)

        @pl.when(j < nb - 1)
        def _():
            out_ref[rows, :] = acc

        @pl.when(j == nb - 1)
        def _():
            o = acc + b2_ref[...]
            m = jnp.max(o, axis=1, keepdims=True)
            e = o - m
            out_ref[rows, :] = e - jnp.log(
                jnp.sum(jnp.exp(e), axis=1, keepdims=True))


@jax.jit
def kernel(feature, adj, W1, b1, W2, b2):
    n, f_in = feature.shape
    h_dim = W1.shape[1]
    c_dim = W2.shape[1]
    nb = -(-n // _BS)
    npad = nb * _BS

    im1, jm1 = np.divmod(np.arange(nb * nb, dtype=np.int32), nb)
    iu, ju = np.triu_indices(nb)
    im = jnp.asarray(np.concatenate([im1, iu.astype(np.int32)]))
    jm = jnp.asarray(np.concatenate([jm1, ju.astype(np.int32)]))
    steps = int(im.shape[0])

    feature_p = jnp.pad(feature, ((0, npad - n), (0, 0)))
    x1 = pl.pallas_call(
        _xw_kernel,
        grid=(nb,),
        in_specs=[
            pl.BlockSpec((_BS, f_in), lambda i: (i, 0)),
            pl.BlockSpec((f_in, h_dim), lambda i: (0, 0)),
        ],
        out_specs=pl.BlockSpec((_BS, h_dim), lambda i: (i, 0)),
        out_shape=jax.ShapeDtypeStruct((npad, h_dim), jnp.bfloat16),
    )(feature_p, W1)

    b1_2d = b1.reshape(1, h_dim)
    b2_2d = b2.reshape(1, c_dim)
    w2_bf = W2.astype(jnp.bfloat16)

    grid_spec = pltpu.PrefetchScalarGridSpec(
        num_scalar_prefetch=2,
        grid=(steps,),
        in_specs=[
            pl.BlockSpec((_BS, _BS), lambda t, im_, jm_: (im_[t], jm_[t])),
            pl.BlockSpec((npad, h_dim), lambda t, im_, jm_: (0, 0)),
            pl.BlockSpec((1, h_dim), lambda t, im_, jm_: (0, 0)),
            pl.BlockSpec((h_dim, c_dim), lambda t, im_, jm_: (0, 0)),
            pl.BlockSpec((1, c_dim), lambda t, im_, jm_: (0, 0)),
        ],
        out_specs=pl.BlockSpec((npad, c_dim), lambda t, im_, jm_: (0, 0)),
        scratch_shapes=[
            pltpu.VMEM((npad, h_dim), jnp.float32),
            pltpu.VMEM((npad, c_dim), jnp.bfloat16),
        ],
    )

    out = pl.pallas_call(
        functools.partial(_tri_kernel, nb=nb, n=n),
        grid_spec=grid_spec,
        out_shape=jax.ShapeDtypeStruct((npad, c_dim), jnp.float32),
    )(im, jm, adj, x1, b1_2d, w2_bf, b2_2d)

    return out[:n]


# speedup vs baseline: 1.0877x; 1.0507x over previous
import functools

import jax
import jax.numpy as jnp
import numpy as np
from jax.experimental import pallas as pl
from jax.experimental.pallas import tpu as pltpu

_BS = 1024


def _xw_kernel(x_ref, w_ref, o_ref):
    o_ref[...] = jnp.dot(x_ref[...], w_ref[...],
                         preferred_element_type=jnp.float32
                         ).astype(jnp.bfloat16)


def _tri_kernel(im_ref, jm_ref, adj_ref, x1_ref, b1_ref, w2_ref, b2_ref,
                out_ref, y_ref, xz_ref, *, nb, n, h_dim):
    t = pl.program_id(0)
    i = im_ref[t]
    j = jm_ref[t]
    sweep1 = t < nb * nb
    rows = pl.ds(i * _BS, _BS)
    kb = n - (nb - 1) * _BS

    @pl.when(j == nb - 1)
    def _():
        adj_ref[:, kb:] = jnp.zeros((_BS, _BS - kb), jnp.float32)

    tile = adj_ref[...].astype(jnp.bfloat16)

    @pl.when(t == 0)
    def _():
        out_ref[...] = jnp.zeros_like(out_ref)
        y_ref[...] = jnp.zeros_like(y_ref)
        xz_ref[:, :h_dim] = x1_ref[...]
        xz_ref[:, h_dim:] = jnp.zeros_like(xz_ref[:, h_dim:])

    @pl.when(sweep1)
    def _():
        p = jnp.dot(tile, xz_ref[pl.ds(j * _BS, _BS), :],
                    preferred_element_type=jnp.float32)
        y_ref[rows, :] = y_ref[rows, :] + p[:, :h_dim]
        out_ref[rows, :] = out_ref[rows, :] + p[:, h_dim:]

        @pl.when(j == nb - 1)
        def _():
            h = jnp.maximum(y_ref[rows, :] + b1_ref[...], 0.0)
            z = jnp.dot(h.astype(jnp.bfloat16), w2_ref[...],
                        preferred_element_type=jnp.float32)
            ridx = i * _BS + jax.lax.broadcasted_iota(jnp.int32, z.shape, 0)
            xz_ref[rows, h_dim:] = jnp.where(ridx < n, z, 0.0
                                             ).astype(jnp.bfloat16)

    @pl.when(jnp.logical_not(sweep1))
    def _():
        acc = out_ref[rows, :] + jnp.dot(
            tile, xz_ref[pl.ds(j * _BS, _BS), h_dim:],
            preferred_element_type=jnp.float32)

        @pl.when(j < nb - 1)
        def _():
            out_ref[rows, :] = acc

        @pl.when(j == nb - 1)
        def _():
            o = acc + b2_ref[...]
            m = jnp.max(o, axis=1, keepdims=True)
            e = o - m
            out_ref[rows, :] = e - jnp.log(
                jnp.sum(jnp.exp(e), axis=1, keepdims=True))


@jax.jit
def kernel(feature, adj, W1, b1, W2, b2):
    n, f_in = feature.shape
    h_dim = W1.shape[1]
    c_dim = W2.shape[1]
    nb = -(-n // _BS)
    npad = nb * _BS

    im1, jm1 = np.divmod(np.arange(nb * nb, dtype=np.int32), nb)
    iu, ju = np.triu_indices(nb)
    im = jnp.asarray(np.concatenate([im1, iu.astype(np.int32)]))
    jm = jnp.asarray(np.concatenate([jm1, ju.astype(np.int32)]))
    steps = int(im.shape[0])

    feature_p = jnp.pad(feature, ((0, npad - n), (0, 0)))
    x1 = pl.pallas_call(
        _xw_kernel,
        grid=(nb,),
        in_specs=[
            pl.BlockSpec((_BS, f_in), lambda i: (i, 0)),
            pl.BlockSpec((f_in, h_dim), lambda i: (0, 0)),
        ],
        out_specs=pl.BlockSpec((_BS, h_dim), lambda i: (i, 0)),
        out_shape=jax.ShapeDtypeStruct((npad, h_dim), jnp.bfloat16),
    )(feature_p, W1)

    b1_2d = b1.reshape(1, h_dim)
    b2_2d = b2.reshape(1, c_dim)
    w2_bf = W2.astype(jnp.bfloat16)

    grid_spec = pltpu.PrefetchScalarGridSpec(
        num_scalar_prefetch=2,
        grid=(steps,),
        in_specs=[
            pl.BlockSpec((_BS, _BS), lambda t, im_, jm_: (im_[t], jm_[t])),
            pl.BlockSpec((npad, h_dim), lambda t, im_, jm_: (0, 0)),
            pl.BlockSpec((1, h_dim), lambda t, im_, jm_: (0, 0)),
            pl.BlockSpec((h_dim, c_dim), lambda t, im_, jm_: (0, 0)),
            pl.BlockSpec((1, c_dim), lambda t, im_, jm_: (0, 0)),
        ],
        out_specs=pl.BlockSpec((npad, c_dim), lambda t, im_, jm_: (0, 0)),
        scratch_shapes=[
            pltpu.VMEM((npad, h_dim), jnp.float32),
            pltpu.VMEM((npad, h_dim + c_dim), jnp.bfloat16),
        ],
    )

    out = pl.pallas_call(
        functools.partial(_tri_kernel, nb=nb, n=n, h_dim=h_dim),
        grid_spec=grid_spec,
        out_shape=jax.ShapeDtypeStruct((npad, c_dim), jnp.float32),
    )(im, jm, adj, x1, b1_2d, w2_bf, b2_2d)

    return out[:n]


# speedup vs baseline: 1.3834x; 1.2719x over previous
import functools

import jax
import jax.numpy as jnp
import numpy as np
from jax.experimental import pallas as pl
from jax.experimental.pallas import tpu as pltpu

_BS = 2048


def _xw_kernel(x_ref, w_ref, o_ref):
    h_dim = w_ref.shape[1]
    o_ref[:, :h_dim] = jnp.dot(x_ref[...], w_ref[...],
                               preferred_element_type=jnp.float32
                               ).astype(jnp.bfloat16)
    o_ref[:, h_dim:] = jnp.zeros_like(o_ref[:, h_dim:])


def _tri_kernel(im_ref, jm_ref, adj_ref, xz_ref, b1_ref, w2_ref, b2_ref,
                out_ref, y_ref, *, nb, n, h_dim):
    t = pl.program_id(0)
    i = im_ref[t]
    j = jm_ref[t]
    sweep1 = t < nb * nb
    rows = pl.ds(i * _BS, _BS)
    kb = n - (nb - 1) * _BS

    @pl.when(j == nb - 1)
    def _():
        adj_ref[:, kb:] = jnp.zeros((_BS, _BS - kb), jnp.float32)

    tile = adj_ref[...].astype(jnp.bfloat16)

    @pl.when(t == 0)
    def _():
        out_ref[...] = jnp.zeros_like(out_ref)
        y_ref[...] = jnp.zeros_like(y_ref)

    @pl.when(sweep1)
    def _():
        p = jnp.dot(tile, xz_ref[pl.ds(j * _BS, _BS), :],
                    preferred_element_type=jnp.float32)
        y_ref[rows, :] = y_ref[rows, :] + p[:, :h_dim]
        out_ref[rows, :] = out_ref[rows, :] + p[:, h_dim:]

        @pl.when(j == nb - 1)
        def _():
            h = jnp.maximum(y_ref[rows, :] + b1_ref[...], 0.0)
            z = jnp.dot(h.astype(jnp.bfloat16), w2_ref[...],
                        preferred_element_type=jnp.float32)
            ridx = i * _BS + jax.lax.broadcasted_iota(jnp.int32, z.shape, 0)
            xz_ref[rows, h_dim:] = jnp.where(ridx < n, z, 0.0
                                             ).astype(jnp.bfloat16)

    @pl.when(jnp.logical_not(sweep1))
    def _():
        acc = out_ref[rows, :] + jnp.dot(
            tile, xz_ref[pl.ds(j * _BS, _BS), h_dim:],
            preferred_element_type=jnp.float32)

        @pl.when(j < nb - 1)
        def _():
            out_ref[rows, :] = acc

        @pl.when(j == nb - 1)
        def _():
            o = acc + b2_ref[...]
            m = jnp.max(o, axis=1, keepdims=True)
            e = o - m
            out_ref[rows, :] = e - jnp.log(
                jnp.sum(jnp.exp(e), axis=1, keepdims=True))


@jax.jit
def kernel(feature, adj, W1, b1, W2, b2):
    n, f_in = feature.shape
    h_dim = W1.shape[1]
    c_dim = W2.shape[1]
    nb = -(-n // _BS)
    npad = nb * _BS

    im1, jm1 = np.divmod(np.arange(nb * nb, dtype=np.int32), nb)
    iu, ju = np.triu_indices(nb)
    im = jnp.asarray(np.concatenate([im1, iu.astype(np.int32)]))
    jm = jnp.asarray(np.concatenate([jm1, ju.astype(np.int32)]))
    steps = int(im.shape[0])

    feature_p = jnp.pad(feature, ((0, npad - n), (0, 0)))
    xz = pl.pallas_call(
        _xw_kernel,
        grid=(nb,),
        in_specs=[
            pl.BlockSpec((_BS, f_in), lambda i: (i, 0)),
            pl.BlockSpec((f_in, h_dim), lambda i: (0, 0)),
        ],
        out_specs=pl.BlockSpec((_BS, h_dim + c_dim), lambda i: (i, 0)),
        out_shape=jax.ShapeDtypeStruct((npad, h_dim + c_dim), jnp.bfloat16),
    )(feature_p, W1)

    b1_2d = b1.reshape(1, h_dim)
    b2_2d = b2.reshape(1, c_dim)
    w2_bf = W2.astype(jnp.bfloat16)

    grid_spec = pltpu.PrefetchScalarGridSpec(
        num_scalar_prefetch=2,
        grid=(steps,),
        in_specs=[
            pl.BlockSpec((_BS, _BS), lambda t, im_, jm_: (im_[t], jm_[t])),
            pl.BlockSpec((npad, h_dim + c_dim), lambda t, im_, jm_: (0, 0)),
            pl.BlockSpec((1, h_dim), lambda t, im_, jm_: (0, 0)),
            pl.BlockSpec((h_dim, c_dim), lambda t, im_, jm_: (0, 0)),
            pl.BlockSpec((1, c_dim), lambda t, im_, jm_: (0, 0)),
        ],
        out_specs=pl.BlockSpec((npad, c_dim), lambda t, im_, jm_: (0, 0)),
        scratch_shapes=[
            pltpu.VMEM((npad, h_dim), jnp.float32),
        ],
    )

    out = pl.pallas_call(
        functools.partial(_tri_kernel, nb=nb, n=n, h_dim=h_dim),
        grid_spec=grid_spec,
        out_shape=jax.ShapeDtypeStruct((npad, c_dim), jnp.float32),
        compiler_params=pltpu.CompilerParams(
            vmem_limit_bytes=100 * 1024 * 1024),
    )(im, jm, adj, xz, b1_2d, w2_bf, b2_2d)

    return out[:n]


# speedup vs baseline: 1.4147x; 1.0226x over previous
import functools

import jax
import jax.numpy as jnp
import numpy as np
from jax.experimental import pallas as pl
from jax.experimental.pallas import tpu as pltpu

_BS = 2048
_HB = _BS // 2


def _tri_kernel(im_ref, jm_ref, adj_t_ref, adj_b_ref, f_ref, w1_ref,
                b1_ref, w2_ref, b2_ref, out_ref, y_ref, xz_ref,
                *, nb, n, h_dim):
    t = pl.program_id(0)
    i = im_ref[t]
    j = jm_ref[t]
    sweep1 = t < nb * nb
    kb = n - (nb - 1) * _BS

    @pl.when(j == nb - 1)
    def _():
        adj_t_ref[:, kb:] = jnp.zeros((_HB, _BS - kb), jnp.float32)
        adj_b_ref[:, kb:] = jnp.zeros((_HB, _BS - kb), jnp.float32)

    @pl.when(t == 0)
    def _():
        out_ref[...] = jnp.zeros_like(out_ref)
        y_ref[...] = jnp.zeros_like(y_ref)
        xz_ref[:, :h_dim] = jnp.dot(
            f_ref[...], w1_ref[...],
            preferred_element_type=jnp.float32).astype(jnp.bfloat16)
        xz_ref[:, h_dim:] = jnp.zeros_like(xz_ref[:, h_dim:])

    tile_t = adj_t_ref[...].astype(jnp.bfloat16)
    tile_b = adj_b_ref[...].astype(jnp.bfloat16)
    rows_t = pl.ds(i * _BS, _HB)
    rows_b = pl.ds(i * _BS + _HB, _HB)

    @pl.when(sweep1)
    def _():
        xzs = xz_ref[pl.ds(j * _BS, _BS), :]
        p_t = jnp.dot(tile_t, xzs, preferred_element_type=jnp.float32)
        p_b = jnp.dot(tile_b, xzs, preferred_element_type=jnp.float32)
        y_ref[rows_t, :] = y_ref[rows_t, :] + p_t[:, :h_dim]
        y_ref[rows_b, :] = y_ref[rows_b, :] + p_b[:, :h_dim]
        out_ref[rows_t, :] = out_ref[rows_t, :] + p_t[:, h_dim:]
        out_ref[rows_b, :] = out_ref[rows_b, :] + p_b[:, h_dim:]

        @pl.when(j == nb - 1)
        def _():
            rows = pl.ds(i * _BS, _BS)
            h = jnp.maximum(y_ref[rows, :] + b1_ref[...], 0.0)
            z = jnp.dot(h.astype(jnp.bfloat16), w2_ref[...],
                        preferred_element_type=jnp.float32)
            ridx = i * _BS + jax.lax.broadcasted_iota(jnp.int32, z.shape, 0)
            xz_ref[rows, h_dim:] = jnp.where(ridx < n, z, 0.0
                                             ).astype(jnp.bfloat16)

    @pl.when(jnp.logical_not(sweep1))
    def _():
        zs = xz_ref[pl.ds(j * _BS, _BS), h_dim:]
        acc_t = out_ref[rows_t, :] + jnp.dot(
            tile_t, zs, preferred_element_type=jnp.float32)
        acc_b = out_ref[rows_b, :] + jnp.dot(
            tile_b, zs, preferred_element_type=jnp.float32)

        @pl.when(j < nb - 1)
        def _():
            out_ref[rows_t, :] = acc_t
            out_ref[rows_b, :] = acc_b

        @pl.when(j == nb - 1)
        def _():
            for acc, rr in ((acc_t, rows_t), (acc_b, rows_b)):
                o = acc + b2_ref[...]
                m = jnp.max(o, axis=1, keepdims=True)
                e = o - m
                out_ref[rr, :] = e - jnp.log(
                    jnp.sum(jnp.exp(e), axis=1, keepdims=True))


@jax.jit
def kernel(feature, adj, W1, b1, W2, b2):
    n, f_in = feature.shape
    h_dim = W1.shape[1]
    c_dim = W2.shape[1]
    nb = -(-n // _BS)
    npad = nb * _BS

    im1, jm1 = np.divmod(np.arange(nb * nb, dtype=np.int32), nb)
    iu, ju = np.triu_indices(nb)
    im = jnp.asarray(np.concatenate([im1, iu.astype(np.int32)]))
    jm = jnp.asarray(np.concatenate([jm1, ju.astype(np.int32)]))
    steps = int(im.shape[0])

    feature_p = jnp.pad(feature.astype(jnp.bfloat16), ((0, npad - n), (0, 0)))
    b1_2d = b1.reshape(1, h_dim)
    b2_2d = b2.reshape(1, c_dim)

    grid_spec = pltpu.PrefetchScalarGridSpec(
        num_scalar_prefetch=2,
        grid=(steps,),
        in_specs=[
            pl.BlockSpec((_HB, _BS),
                         lambda t, im_, jm_: (2 * im_[t], jm_[t])),
            pl.BlockSpec((_HB, _BS),
                         lambda t, im_, jm_: (2 * im_[t] + 1, jm_[t])),
            pl.BlockSpec((npad, f_in), lambda t, im_, jm_: (0, 0)),
            pl.BlockSpec((f_in, h_dim), lambda t, im_, jm_: (0, 0)),
            pl.BlockSpec((1, h_dim), lambda t, im_, jm_: (0, 0)),
            pl.BlockSpec((h_dim, c_dim), lambda t, im_, jm_: (0, 0)),
            pl.BlockSpec((1, c_dim), lambda t, im_, jm_: (0, 0)),
        ],
        out_specs=pl.BlockSpec((npad, c_dim), lambda t, im_, jm_: (0, 0)),
        scratch_shapes=[
            pltpu.VMEM((npad, h_dim), jnp.float32),
            pltpu.VMEM((npad, h_dim + c_dim), jnp.bfloat16),
        ],
    )

    out = pl.pallas_call(
        functools.partial(_tri_kernel, nb=nb, n=n, h_dim=h_dim),
        grid_spec=grid_spec,
        out_shape=jax.ShapeDtypeStruct((npad, c_dim), jnp.float32),
        compiler_params=pltpu.CompilerParams(
            vmem_limit_bytes=100 * 1024 * 1024),
    )(im, jm, adj, adj, feature_p, W1.astype(jnp.bfloat16), b1_2d,
      W2.astype(jnp.bfloat16), b2_2d)

    return out[:n]


# speedup vs baseline: 1.4969x; 1.0581x over previous
import functools

import jax
import jax.numpy as jnp
import numpy as np
from jax.experimental import pallas as pl
from jax.experimental.pallas import tpu as pltpu

_BS = 2048
_HB = _BS // 2


def _tri_kernel(im_ref, jm_ref, adj_t_ref, adj_b_ref, f_ref, w1_ref,
                b1_ref, w2_ref, b2_ref, out_ref, y_ref, xz_ref,
                *, nb, n, h_dim):
    t = pl.program_id(0)
    i = im_ref[t]
    j = jm_ref[t]
    sweep1 = t < nb * nb
    kb = n - (nb - 1) * _BS

    @pl.when(j == nb - 1)
    def _():
        adj_t_ref[:, kb:] = jnp.zeros((_HB, _BS - kb), jnp.float32)
        adj_b_ref[:, kb:] = jnp.zeros((_HB, _BS - kb), jnp.float32)

    @pl.when(t == 0)
    def _():
        out_ref[...] = jnp.zeros_like(out_ref)
        y_ref[...] = jnp.zeros_like(y_ref)
        xz_ref[:, :h_dim] = jnp.dot(
            f_ref[...], w1_ref[...],
            preferred_element_type=jnp.float32).astype(jnp.bfloat16)
        xz_ref[:, h_dim:] = jnp.zeros_like(xz_ref[:, h_dim:])

    tile_t = adj_t_ref[...].astype(jnp.bfloat16)
    tile_b = adj_b_ref[...].astype(jnp.bfloat16)
    rows_t = pl.ds(i * _BS, _HB)
    rows_b = pl.ds(i * _BS + _HB, _HB)

    @pl.when(sweep1)
    def _():
        xzs = xz_ref[pl.ds(j * _BS, _BS), :]
        p_t = jnp.dot(tile_t, xzs, preferred_element_type=jnp.float32)
        p_b = jnp.dot(tile_b, xzs, preferred_element_type=jnp.float32)
        y_ref[rows_t, :] = y_ref[rows_t, :] + p_t[:, :h_dim]
        y_ref[rows_b, :] = y_ref[rows_b, :] + p_b[:, :h_dim]
        out_ref[rows_t, :] = out_ref[rows_t, :] + p_t[:, h_dim:]
        out_ref[rows_b, :] = out_ref[rows_b, :] + p_b[:, h_dim:]

        @pl.when(j == i)
        def _():
            rows = pl.ds(i * _BS, _BS)
            h = jnp.maximum(y_ref[rows, :] + b1_ref[...], 0.0)
            z = jnp.dot(h.astype(jnp.bfloat16), w2_ref[...],
                        preferred_element_type=jnp.float32)
            ridx = i * _BS + jax.lax.broadcasted_iota(jnp.int32, z.shape, 0)
            zb = jnp.where(ridx < n, z, 0.0).astype(jnp.bfloat16)
            xz_ref[rows, h_dim:] = zb
            d_t = out_ref[rows_t, :] + jnp.dot(
                tile_t, zb, preferred_element_type=jnp.float32)
            d_b = out_ref[rows_b, :] + jnp.dot(
                tile_b, zb, preferred_element_type=jnp.float32)

            @pl.when(i < nb - 1)
            def _():
                out_ref[rows_t, :] = d_t
                out_ref[rows_b, :] = d_b

            @pl.when(i == nb - 1)
            def _():
                for acc, rr in ((d_t, rows_t), (d_b, rows_b)):
                    o = acc + b2_ref[...]
                    m = jnp.max(o, axis=1, keepdims=True)
                    e = o - m
                    out_ref[rr, :] = e - jnp.log(
                        jnp.sum(jnp.exp(e), axis=1, keepdims=True))

    @pl.when(jnp.logical_not(sweep1))
    def _():
        zs = xz_ref[pl.ds(j * _BS, _BS), h_dim:]
        acc_t = out_ref[rows_t, :] + jnp.dot(
            tile_t, zs, preferred_element_type=jnp.float32)
        acc_b = out_ref[rows_b, :] + jnp.dot(
            tile_b, zs, preferred_element_type=jnp.float32)

        @pl.when(j < nb - 1)
        def _():
            out_ref[rows_t, :] = acc_t
            out_ref[rows_b, :] = acc_b

        @pl.when(j == nb - 1)
        def _():
            for acc, rr in ((acc_t, rows_t), (acc_b, rows_b)):
                o = acc + b2_ref[...]
                m = jnp.max(o, axis=1, keepdims=True)
                e = o - m
                out_ref[rr, :] = e - jnp.log(
                    jnp.sum(jnp.exp(e), axis=1, keepdims=True))


@jax.jit
def kernel(feature, adj, W1, b1, W2, b2):
    n, f_in = feature.shape
    h_dim = W1.shape[1]
    c_dim = W2.shape[1]
    nb = -(-n // _BS)
    npad = nb * _BS

    im_l, jm_l = [], []
    for i in range(nb):
        for j in range(nb):
            if j != i:
                im_l.append(i)
                jm_l.append(j)
        im_l.append(i)
        jm_l.append(i)
    for i in range(nb):
        for j in range(i + 1, nb):
            im_l.append(i)
            jm_l.append(j)
    im = jnp.asarray(np.asarray(im_l, dtype=np.int32))
    jm = jnp.asarray(np.asarray(jm_l, dtype=np.int32))
    steps = int(im.shape[0])

    feature_p = jnp.pad(feature.astype(jnp.bfloat16), ((0, npad - n), (0, 0)))
    b1_2d = b1.reshape(1, h_dim)
    b2_2d = b2.reshape(1, c_dim)

    grid_spec = pltpu.PrefetchScalarGridSpec(
        num_scalar_prefetch=2,
        grid=(steps,),
        in_specs=[
            pl.BlockSpec((_HB, _BS),
                         lambda t, im_, jm_: (2 * im_[t], jm_[t])),
            pl.BlockSpec((_HB, _BS),
                         lambda t, im_, jm_: (2 * im_[t] + 1, jm_[t])),
            pl.BlockSpec((npad, f_in), lambda t, im_, jm_: (0, 0)),
            pl.BlockSpec((f_in, h_dim), lambda t, im_, jm_: (0, 0)),
            pl.BlockSpec((1, h_dim), lambda t, im_, jm_: (0, 0)),
            pl.BlockSpec((h_dim, c_dim), lambda t, im_, jm_: (0, 0)),
            pl.BlockSpec((1, c_dim), lambda t, im_, jm_: (0, 0)),
        ],
        out_specs=pl.BlockSpec((npad, c_dim), lambda t, im_, jm_: (0, 0)),
        scratch_shapes=[
            pltpu.VMEM((npad, h_dim), jnp.float32),
            pltpu.VMEM((npad, h_dim + c_dim), jnp.bfloat16),
        ],
    )

    out = pl.pallas_call(
        functools.partial(_tri_kernel, nb=nb, n=n, h_dim=h_dim),
        grid_spec=grid_spec,
        out_shape=jax.ShapeDtypeStruct((npad, c_dim), jnp.float32),
        compiler_params=pltpu.CompilerParams(
            vmem_limit_bytes=100 * 1024 * 1024),
    )(im, jm, adj, adj, feature_p, W1.astype(jnp.bfloat16), b1_2d,
      W2.astype(jnp.bfloat16), b2_2d)

    return out[:n]
